# Initial kernel scaffold; baseline (speedup 1.0000x reference)
import jax
import jax.numpy as jnp
from jax.experimental import pallas as pl


def kernel(batch, predicted_durations):
    raise NotImplementedError("write your pallas kernel here")


# speedup vs baseline: 20.4459x; 20.4459x over previous
import functools

import jax
import jax.numpy as jnp
from jax import lax
from jax.experimental import pallas as pl
from jax.experimental.pallas import tpu as pltpu
from jax.experimental.pallas import tpu_sc as plsc

B = 8
T = 512
D = 384
F = 1600
TBL = B * T
PAD_ROWS = 8
ZERO_ROW = TBL

NC, NS = 2, 16
NW = NC * NS
ROWS_PER_W = (B * F) // NW
CHUNK = 80
NCH = ROWS_PER_W // CHUNK


def _idx_body(pd_ref, idx_ref):
    b = pl.program_id(0)
    d = jnp.round(jnp.clip(pd_ref[...], 1.0, 20.0)).reshape(1, T)
    rows = lax.broadcasted_iota(jnp.int32, (T, T), 0)
    cols = lax.broadcasted_iota(jnp.int32, (T, T), 1)
    tri = (rows <= cols).astype(jnp.float32)
    cum = jnp.dot(d, tri, preferred_element_type=jnp.float32)
    pos = lax.broadcasted_iota(jnp.int32, (F, T), 0).astype(jnp.float32)
    m = (pos >= cum).astype(jnp.float32)
    tok = jnp.dot(m, jnp.ones((T, 1), jnp.float32),
                  preferred_element_type=jnp.float32)
    raw = tok.astype(jnp.int32)
    flat = jnp.where(raw < T, b * T + raw, ZERO_ROW)
    idx_ref[...] = flat.reshape(1, F, 1)


_idx_call = pl.pallas_call(
    _idx_body,
    grid=(B,),
    in_specs=[pl.BlockSpec((1, 1, T), lambda b: (b, 0, 0))],
    out_specs=pl.BlockSpec((1, F, 1), lambda b: (b, 0, 0)),
    out_shape=jax.ShapeDtypeStruct((B, F, 1), jnp.int32),
)


_sc_mesh = plsc.VectorSubcoreMesh(core_axis_name="c", subcore_axis_name="s")


@functools.partial(
    pl.kernel,
    mesh=_sc_mesh,
    out_type=jax.ShapeDtypeStruct((B * F, D), jnp.float32),
    scratch_types=[
        pltpu.VMEM((ROWS_PER_W,), jnp.int32),
        pltpu.VMEM((CHUNK, D), jnp.float32),
        pltpu.VMEM((CHUNK, D), jnp.float32),
        pltpu.SemaphoreType.DMA,
        pltpu.SemaphoreType.DMA,
    ],
)
def _sc_gather(table_hbm, idx_hbm, out_hbm, idx_v, buf0, buf1, sem0, sem1):
    wid = lax.axis_index("s") * NC + lax.axis_index("c")
    base = wid * ROWS_PER_W
    pltpu.sync_copy(idx_hbm.at[pl.ds(base, ROWS_PER_W)], idx_v)
    bufs = (buf0, buf1)
    sems = (sem0, sem1)
    cps = [None, None]
    cps[0] = pltpu.async_copy(
        table_hbm.at[idx_v.at[pl.ds(0, CHUNK)]], buf0, sem0)
    for c in range(NCH):
        nxt = c + 1
        if nxt < NCH:
            cps[nxt % 2] = pltpu.async_copy(
                table_hbm.at[idx_v.at[pl.ds(nxt * CHUNK, CHUNK)]],
                bufs[nxt % 2], sems[nxt % 2])
        cps[c % 2].wait()
        pltpu.sync_copy(bufs[c % 2],
                        out_hbm.at[pl.ds(base + c * CHUNK, CHUNK)])


def kernel(batch, predicted_durations):
    pd = predicted_durations.reshape(B, 1, T)
    idx = _idx_call(pd)
    table = jnp.concatenate(
        [batch.reshape(TBL, D), jnp.zeros((PAD_ROWS, D), jnp.float32)], axis=0)
    out = _sc_gather(table, idx.reshape(B * F))
    return out.reshape(B, F, D)
